# baseline (speedup 1.0000x reference)
import jax
import jax.numpy as jnp
from jax import lax
from jax.experimental import pallas as pl
from jax.experimental.pallas import tpu as pltpu
from jax.experimental.pallas import tpu_sc as plsc

N_USERS = 25000
N_ITEMS = 25000
N = N_USERS + N_ITEMS
EMB = 64
HALF = EMB // 2
N_LAYERS = 3
NE = 800000

NC = 2
NS = 16
EB = 128
NBLK = 392
NE_PAD = NS * NBLK * EB
NBLK_D = NBLK // NC
SUBROWS = 3200
A = NS * SUBROWS
PAD_COL = A - 8
K = 28
NCHUNK = NBLK // K

_mesh = plsc.VectorSubcoreMesh(core_axis_name="c", subcore_axis_name="s")
_sc_params = pltpu.CompilerParams(use_tc_tiling_on_sc=False)


def _deg_body(c_hbm, out_hbm, acc, cidx, buf):
    c = lax.axis_index("c")
    s = lax.axis_index("s")

    for i in range(0, EB, 16):
        buf[pl.ds(i, 16)] = jnp.zeros((16,), jnp.float32)

    @pl.loop(0, SUBROWS, step=EB)
    def _(r):
        pltpu.sync_copy(buf, acc.at[pl.ds(s * SUBROWS + r, EB)])

    plsc.subcore_barrier()

    for i in range(0, EB, 16):
        buf[pl.ds(i, 16)] = jnp.ones((16,), jnp.float32)

    pltpu.sync_copy(c_hbm.at[pl.ds((c * NS + s) * NBLK_D, NBLK_D)], cidx)

    @pl.loop(0, NBLK_D)
    def _(j):
        pltpu.sync_copy(buf, acc.at[cidx.at[j]], add=True)

    plsc.subcore_barrier()
    pltpu.sync_copy(acc.at[pl.ds(s * SUBROWS, SUBROWS)],
                    out_hbm.at[pl.ds(c * A + s * SUBROWS, SUBROWS)])


_deg_call = pl.kernel(
    _deg_body,
    out_type=jax.ShapeDtypeStruct((NC * A,), jnp.float32),
    mesh=_mesh,
    scratch_types=[
        pltpu.VMEM_SHARED((A,), jnp.float32),
        pltpu.VMEM((NBLK_D, EB), jnp.int32),
        pltpu.VMEM((EB,), jnp.float32),
    ],
    compiler_params=_sc_params,
)


def _layer_body(xs2_hbm, g_hbm, c_hbm, out_hbm, acc, gch, cch, rows,
                sg0, sg1, si0, si1):
    c = lax.axis_index("c")
    s = lax.axis_index("s")
    gbase = (c * NS + s) * NBLK
    cbase = s * NBLK

    for r in range(EB):
        for k in range(0, HALF, 16):
            rows[0, r, pl.ds(k, 16)] = jnp.zeros((16,), jnp.float32)

    @pl.loop(0, SUBROWS, step=EB)
    def _(r):
        pltpu.sync_copy(rows.at[0], acc.at[pl.ds(s * SUBROWS + r, EB)])

    plsc.subcore_barrier()

    pltpu.async_copy(g_hbm.at[pl.ds(gbase, K)], gch.at[0], si0)
    pltpu.async_copy(c_hbm.at[pl.ds(cbase, K)], cch.at[0], si0)
    pltpu.async_copy(g_hbm.at[pl.ds(gbase + K, K)], gch.at[1], si1)
    pltpu.async_copy(c_hbm.at[pl.ds(cbase + K, K)], cch.at[1], si1)

    def chunk(i, slot, si_this):
        pltpu.make_async_copy(g_hbm.at[pl.ds(gbase, K)], gch.at[slot], si_this).wait()
        pltpu.make_async_copy(c_hbm.at[pl.ds(cbase, K)], cch.at[slot], si_this).wait()

        pltpu.async_copy(xs2_hbm.at[gch.at[slot, 0]], rows.at[0], sg0)

        @pl.loop(0, K, step=2)
        def _(k):
            pltpu.async_copy(xs2_hbm.at[gch.at[slot, k + 1]], rows.at[1], sg1)
            pltpu.make_async_copy(xs2_hbm.at[gch.at[slot, k]], rows.at[0], sg0).wait()
            pltpu.sync_copy(rows.at[0], acc.at[cch.at[slot, k]], add=True)

            @pl.when(k + 2 < K)
            def _():
                pltpu.async_copy(xs2_hbm.at[gch.at[slot, k + 2]], rows.at[0], sg0)

            pltpu.make_async_copy(xs2_hbm.at[gch.at[slot, k + 1]], rows.at[1], sg1).wait()
            pltpu.sync_copy(rows.at[1], acc.at[cch.at[slot, k + 1]], add=True)

        @pl.when(i + 2 < NCHUNK)
        def _():
            pltpu.async_copy(g_hbm.at[pl.ds(gbase + (i + 2) * K, K)], gch.at[slot], si_this)
            pltpu.async_copy(c_hbm.at[pl.ds(cbase + (i + 2) * K, K)], cch.at[slot], si_this)

    @pl.loop(0, NCHUNK, step=2)
    def _(i):
        chunk(i, 0, si0)
        chunk(i + 1, 1, si1)

    plsc.subcore_barrier()
    pltpu.sync_copy(acc.at[pl.ds(s * SUBROWS, SUBROWS)],
                    out_hbm.at[pl.ds(c * A + s * SUBROWS, SUBROWS)])


_layer_call = pl.kernel(
    _layer_body,
    out_type=jax.ShapeDtypeStruct((NC * A, HALF), jnp.float32),
    mesh=_mesh,
    scratch_types=[
        pltpu.VMEM_SHARED((A, HALF), jnp.float32),
        pltpu.VMEM((2, K, EB), jnp.int32),
        pltpu.VMEM((2, K, EB), jnp.int32),
        pltpu.VMEM((2, EB, HALF), jnp.float32),
        pltpu.SemaphoreType.DMA,
        pltpu.SemaphoreType.DMA,
        pltpu.SemaphoreType.DMA,
        pltpu.SemaphoreType.DMA,
    ],
    compiler_params=_sc_params,
)


_R = 1600
_G = 32


def _pre_tc(d0_ref, d1_ref, x_ref, dis_ref, t_ref):
    d = d0_ref[...] + d1_ref[...]
    dis = jnp.where(d > 0.0, lax.rsqrt(d), 0.0)
    dis_ref[...] = dis
    xs = x_ref[...] * dis
    t_ref[0] = xs[:, :HALF]
    t_ref[1] = xs[:, HALF:]


def _mid_tc(a0_ref, a1_ref, dis_ref, t_ref):
    dis = dis_ref[...]
    d2 = dis * dis
    t_ref[0] = a0_ref[...] * d2
    t_ref[1] = a1_ref[...] * d2


def _fin_tc(b10_ref, b11_ref, b20_ref, b21_ref, b30_ref, b31_ref,
            x0_ref, dis_ref, x1_ref, x2_ref, x3_ref, mean_ref):
    dis = dis_ref[...]
    x0 = x0_ref[...]
    x1 = jnp.concatenate([b10_ref[...], b11_ref[...]], axis=1) * dis
    x2 = jnp.concatenate([b20_ref[...], b21_ref[...]], axis=1) * dis
    x3 = jnp.concatenate([b30_ref[...], b31_ref[...]], axis=1) * dis
    x1_ref[...] = x1
    x2_ref[...] = x2
    x3_ref[...] = x3
    mean_ref[...] = (x0 + x1 + x2 + x3) * 0.25


def _spec(w):
    return pl.BlockSpec((_R, w), lambda i: (i, 0))


def _spec2(w):
    return pl.BlockSpec((_R, w), lambda i: (i + _G, 0))


_t_spec = pl.BlockSpec((NC, _R, HALF), lambda i: (0, i, 0))
_t_type = jax.ShapeDtypeStruct((NC, A, HALF), jnp.float32)

_pre_call = pl.pallas_call(
    _pre_tc,
    grid=(_G,),
    in_specs=[_spec(1), _spec2(1), _spec(EMB)],
    out_specs=[_spec(1), _t_spec],
    out_shape=[jax.ShapeDtypeStruct((N, 1), jnp.float32), _t_type],
)

_mid_call = pl.pallas_call(
    _mid_tc,
    grid=(_G,),
    in_specs=[_spec(HALF), _spec2(HALF), _spec(1)],
    out_specs=_t_spec,
    out_shape=_t_type,
)

_fin_call = pl.pallas_call(
    _fin_tc,
    grid=(_G,),
    in_specs=[_spec(HALF), _spec2(HALF),
              _spec(HALF), _spec2(HALF),
              _spec(HALF), _spec2(HALF),
              _spec(EMB), _spec(1)],
    out_specs=[_spec(EMB), _spec(EMB), _spec(EMB), _spec(EMB)],
    out_shape=[jax.ShapeDtypeStruct((N, EMB), jnp.float32),
               jax.ShapeDtypeStruct((N, EMB), jnp.float32),
               jax.ShapeDtypeStruct((N, EMB), jnp.float32),
               jax.ShapeDtypeStruct((N, EMB), jnp.float32)],
)


def kernel(user_emb, item_emb, edge_index):
    x0 = jnp.concatenate([user_emb, item_emb], axis=0)
    row = edge_index[0].astype(jnp.int32)
    col = edge_index[1].astype(jnp.int32)
    pad = NE_PAD - NE
    rowp = jnp.concatenate([row, jnp.zeros((pad,), jnp.int32)])
    colp = jnp.concatenate([col, jnp.full((pad,), PAD_COL, jnp.int32)])
    colb = colp.reshape(NS * NBLK, EB)
    gb = jnp.concatenate([rowp, rowp + A]).reshape(NC * NS * NBLK, EB)

    degp = _deg_call(colb)[:, None]
    dis, t = _pre_call(degp, degp, x0)

    outs = []
    for l in range(N_LAYERS):
        outf = _layer_call(t.reshape(NC * A, HALF), gb, colb)
        outs.append(outf)
        if l < N_LAYERS - 1:
            t = _mid_call(outf, outf, dis)

    x1, x2, x3, mean = _fin_call(outs[0], outs[0], outs[1], outs[1],
                                 outs[2], outs[2], x0, dis)
    all_emb = jnp.stack([x0, x1, x2, x3], axis=1)
    return (mean[:N_USERS], mean[N_ITEMS:], all_emb)


# speedup vs baseline: 1.0084x; 1.0084x over previous
import jax
import jax.numpy as jnp
from jax import lax
from jax.experimental import pallas as pl
from jax.experimental.pallas import tpu as pltpu
from jax.experimental.pallas import tpu_sc as plsc

N_USERS = 25000
N_ITEMS = 25000
N = N_USERS + N_ITEMS
EMB = 64
HALF = EMB // 2
N_LAYERS = 3
NE = 800000

NC = 2
NS = 16
EB = 128
NBLK = 392
NE_PAD = NS * NBLK * EB
NBLK_D = NBLK // NC
SUBROWS = 3200
A = NS * SUBROWS
PAD_COL = A - 8
K = 28
NCHUNK = NBLK // K

_mesh = plsc.VectorSubcoreMesh(core_axis_name="c", subcore_axis_name="s")
_sc_params = pltpu.CompilerParams(use_tc_tiling_on_sc=False)


def _deg_body(c_hbm, out_hbm, acc, cidx, buf):
    c = lax.axis_index("c")
    s = lax.axis_index("s")

    for i in range(0, EB, 16):
        buf[pl.ds(i, 16)] = jnp.zeros((16,), jnp.float32)

    @pl.loop(0, SUBROWS, step=EB)
    def _(r):
        pltpu.sync_copy(buf, acc.at[pl.ds(s * SUBROWS + r, EB)])

    plsc.subcore_barrier()

    for i in range(0, EB, 16):
        buf[pl.ds(i, 16)] = jnp.ones((16,), jnp.float32)

    pltpu.sync_copy(c_hbm.at[pl.ds((c * NS + s) * NBLK_D, NBLK_D)], cidx)

    @pl.loop(0, NBLK_D)
    def _(j):
        pltpu.sync_copy(buf, acc.at[cidx.at[j]], add=True)

    plsc.subcore_barrier()
    pltpu.sync_copy(acc.at[pl.ds(s * SUBROWS, SUBROWS)],
                    out_hbm.at[pl.ds(c * A + s * SUBROWS, SUBROWS)])


_deg_call = pl.kernel(
    _deg_body,
    out_type=jax.ShapeDtypeStruct((NC * A,), jnp.float32),
    mesh=_mesh,
    scratch_types=[
        pltpu.VMEM_SHARED((A,), jnp.float32),
        pltpu.VMEM((NBLK_D, EB), jnp.int32),
        pltpu.VMEM((EB,), jnp.float32),
    ],
    compiler_params=_sc_params,
)


def _layer_body(xs2_hbm, g_hbm, c_hbm, out_hbm, acc, gch, cch, rows,
                sg0, sg1, si0, si1):
    c = lax.axis_index("c")
    s = lax.axis_index("s")
    gbase = (c * NS + s) * NBLK
    cbase = s * NBLK

    for r in range(EB):
        for k in range(0, HALF, 16):
            rows[0, r, pl.ds(k, 16)] = jnp.zeros((16,), jnp.float32)

    @pl.loop(0, SUBROWS, step=EB)
    def _(r):
        pltpu.sync_copy(rows.at[0], acc.at[pl.ds(s * SUBROWS + r, EB)])

    plsc.subcore_barrier()

    def wait_gather(b, sem):
        pltpu.make_async_copy(xs2_hbm.at[gch.at[0, 0]], rows.at[b], sem).wait()

    pltpu.async_copy(g_hbm.at[pl.ds(gbase, K + 2)], gch.at[0], si0)
    pltpu.async_copy(c_hbm.at[pl.ds(cbase, K + 2)], cch.at[0], si0)
    pltpu.make_async_copy(g_hbm.at[pl.ds(gbase, K + 2)], gch.at[0], si0).wait()
    pltpu.make_async_copy(c_hbm.at[pl.ds(cbase, K + 2)], cch.at[0], si0).wait()
    pltpu.async_copy(xs2_hbm.at[gch.at[0, 0]], rows.at[0], sg0)

    def chunk(i, slot, si_this, si_other):
        @pl.when(i > 0)
        def _():
            pltpu.make_async_copy(g_hbm.at[pl.ds(gbase, K + 2)], gch.at[slot], si_this).wait()
            pltpu.make_async_copy(c_hbm.at[pl.ds(cbase, K + 2)], cch.at[slot], si_this).wait()

        pltpu.async_copy(xs2_hbm.at[gch.at[slot, 1]], rows.at[1], sg1)
        wait_gather(0, sg0)
        pltpu.sync_copy(rows.at[0], acc.at[cch.at[slot, 0]], add=True)

        @pl.when(i + 1 < NCHUNK)
        def _():
            pltpu.async_copy(g_hbm.at[pl.ds(gbase + (i + 1) * K, K + 2)],
                             gch.at[1 - slot], si_other)
            pltpu.async_copy(c_hbm.at[pl.ds(cbase + (i + 1) * K, K + 2)],
                             cch.at[1 - slot], si_other)

        pltpu.async_copy(xs2_hbm.at[gch.at[slot, 2]], rows.at[0], sg0)
        wait_gather(1, sg1)
        pltpu.sync_copy(rows.at[1], acc.at[cch.at[slot, 1]], add=True)

        @pl.loop(2, K, step=2)
        def _(k):
            pltpu.async_copy(xs2_hbm.at[gch.at[slot, k + 1]], rows.at[1], sg1)
            wait_gather(0, sg0)
            pltpu.sync_copy(rows.at[0], acc.at[cch.at[slot, k]], add=True)
            pltpu.async_copy(xs2_hbm.at[gch.at[slot, k + 2]], rows.at[0], sg0)
            wait_gather(1, sg1)
            pltpu.sync_copy(rows.at[1], acc.at[cch.at[slot, k + 1]], add=True)

    @pl.loop(0, NCHUNK, step=2)
    def _(i):
        chunk(i, 0, si0, si1)
        chunk(i + 1, 1, si1, si0)

    wait_gather(0, sg0)
    plsc.subcore_barrier()
    pltpu.sync_copy(acc.at[pl.ds(s * SUBROWS, SUBROWS)],
                    out_hbm.at[pl.ds(c * A + s * SUBROWS, SUBROWS)])


_layer_call = pl.kernel(
    _layer_body,
    out_type=jax.ShapeDtypeStruct((NC * A, HALF), jnp.float32),
    mesh=_mesh,
    scratch_types=[
        pltpu.VMEM_SHARED((A, HALF), jnp.float32),
        pltpu.VMEM((2, K + 2, EB), jnp.int32),
        pltpu.VMEM((2, K + 2, EB), jnp.int32),
        pltpu.VMEM((2, EB, HALF), jnp.float32),
        pltpu.SemaphoreType.DMA,
        pltpu.SemaphoreType.DMA,
        pltpu.SemaphoreType.DMA,
        pltpu.SemaphoreType.DMA,
    ],
    compiler_params=_sc_params,
)


_R = 1600
_G = 32


def _pre_tc(d0_ref, d1_ref, x_ref, dis_ref, t_ref):
    d = d0_ref[...] + d1_ref[...]
    dis = jnp.where(d > 0.0, lax.rsqrt(d), 0.0)
    dis_ref[...] = dis
    xs = x_ref[...] * dis
    t_ref[0] = xs[:, :HALF]
    t_ref[1] = xs[:, HALF:]


def _mid_tc(a0_ref, a1_ref, dis_ref, t_ref):
    dis = dis_ref[...]
    d2 = dis * dis
    t_ref[0] = a0_ref[...] * d2
    t_ref[1] = a1_ref[...] * d2


def _fin_tc(b10_ref, b11_ref, b20_ref, b21_ref, b30_ref, b31_ref,
            x0_ref, dis_ref, x1_ref, x2_ref, x3_ref, mean_ref):
    dis = dis_ref[...]
    x0 = x0_ref[...]
    x1 = jnp.concatenate([b10_ref[...], b11_ref[...]], axis=1) * dis
    x2 = jnp.concatenate([b20_ref[...], b21_ref[...]], axis=1) * dis
    x3 = jnp.concatenate([b30_ref[...], b31_ref[...]], axis=1) * dis
    x1_ref[...] = x1
    x2_ref[...] = x2
    x3_ref[...] = x3
    mean_ref[...] = (x0 + x1 + x2 + x3) * 0.25


def _spec(w):
    return pl.BlockSpec((_R, w), lambda i: (i, 0))


def _spec2(w):
    return pl.BlockSpec((_R, w), lambda i: (i + _G, 0))


_t_spec = pl.BlockSpec((NC, _R, HALF), lambda i: (0, i, 0))
_t_type = jax.ShapeDtypeStruct((NC, A, HALF), jnp.float32)

_pre_call = pl.pallas_call(
    _pre_tc,
    grid=(_G,),
    in_specs=[_spec(1), _spec2(1), _spec(EMB)],
    out_specs=[_spec(1), _t_spec],
    out_shape=[jax.ShapeDtypeStruct((N, 1), jnp.float32), _t_type],
)

_mid_call = pl.pallas_call(
    _mid_tc,
    grid=(_G,),
    in_specs=[_spec(HALF), _spec2(HALF), _spec(1)],
    out_specs=_t_spec,
    out_shape=_t_type,
)

_fin_call = pl.pallas_call(
    _fin_tc,
    grid=(_G,),
    in_specs=[_spec(HALF), _spec2(HALF),
              _spec(HALF), _spec2(HALF),
              _spec(HALF), _spec2(HALF),
              _spec(EMB), _spec(1)],
    out_specs=[_spec(EMB), _spec(EMB), _spec(EMB), _spec(EMB)],
    out_shape=[jax.ShapeDtypeStruct((N, EMB), jnp.float32),
               jax.ShapeDtypeStruct((N, EMB), jnp.float32),
               jax.ShapeDtypeStruct((N, EMB), jnp.float32),
               jax.ShapeDtypeStruct((N, EMB), jnp.float32)],
)


def kernel(user_emb, item_emb, edge_index):
    x0 = jnp.concatenate([user_emb, item_emb], axis=0)
    row = edge_index[0].astype(jnp.int32)
    col = edge_index[1].astype(jnp.int32)
    pad = NE_PAD - NE
    rowp = jnp.concatenate([row, jnp.zeros((pad,), jnp.int32)])
    colp = jnp.concatenate([col, jnp.full((pad,), PAD_COL, jnp.int32)])
    colb = colp.reshape(NS * NBLK, EB)
    gb = jnp.concatenate([rowp, rowp + A]).reshape(NC * NS * NBLK, EB)
    zrows = jnp.zeros((2, EB), jnp.int32)
    colb = jnp.concatenate([colb, zrows])
    gb = jnp.concatenate([gb, zrows])

    degp = _deg_call(colb)[:, None]
    dis, t = _pre_call(degp, degp, x0)

    outs = []
    for l in range(N_LAYERS):
        outf = _layer_call(t.reshape(NC * A, HALF), gb, colb)
        outs.append(outf)
        if l < N_LAYERS - 1:
            t = _mid_call(outf, outf, dis)

    x1, x2, x3, mean = _fin_call(outs[0], outs[0], outs[1], outs[1],
                                 outs[2], outs[2], x0, dis)
    all_emb = jnp.stack([x0, x1, x2, x3], axis=1)
    return (mean[:N_USERS], mean[N_ITEMS:], all_emb)
